# baseline (speedup 1.0000x reference)
import functools

import jax
import jax.numpy as jnp
from jax import lax
from jax.experimental import pallas as pl
from jax.experimental.pallas import tpu as pltpu
from jax.experimental.pallas import tpu_sc as plsc

_NC = 2
_NS = 16
_NW = _NC * _NS
_L = 16


def _trig_body(rel_ref, cs_ref):
    hd = rel_ref.shape[1]
    x = rel_ref[...]
    cs_ref[:, 0:hd] = jnp.cos(x)
    cs_ref[:, hd:2 * hd] = jnp.sin(x)


def _score_body(C, D, ROWS, NCHUNK,
                h_hbm, r_hbm, t_hbm, ent_hbm, cs_hbm, out_hbm,
                hidx, ridx, tidx, hrows, trows, csrows, outv, sem0, sem1):
    HD = D // 2
    G = HD // _L
    wid = lax.axis_index("s") * _NC + lax.axis_index("c")
    base = wid * ROWS

    for k in range(NCHUNK):
        pltpu.sync_copy(h_hbm.at[pl.ds(base + k * C, C)], hidx.at[k])
        pltpu.sync_copy(t_hbm.at[pl.ds(base + k * C, C)], tidx.at[k])
        pltpu.sync_copy(r_hbm.at[pl.ds(base + k * C, C)], ridx.at[k])

    sems = (sem0, sem1)

    def issue(k):
        b = k % 2
        return (pltpu.async_copy(ent_hbm.at[hidx.at[k]], hrows.at[b], sems[b]),
                pltpu.async_copy(ent_hbm.at[tidx.at[k]], trows.at[b], sems[b]),
                pltpu.async_copy(cs_hbm.at[ridx.at[k]], csrows.at[b], sems[b]))

    pending = issue(0)
    for k in range(NCHUNK):
        b = k % 2
        nxt = issue(k + 1) if k + 1 < NCHUNK else None
        for cp in pending:
            cp.wait()
        pending = nxt

        lane_iota = lax.broadcasted_iota(jnp.int32, (_L,), 0)

        @plsc.parallel_loop(0, C // _L, unroll=2)
        def block_fn(jb):
            i0 = jb * _L
            blockvec = jnp.zeros((_L,), jnp.float32)
            for u in range(_L):
                i = i0 + u
                acc = None
                for g in range(G):
                    hr = hrows[b, i, pl.ds(g * _L, _L)]
                    hi = hrows[b, i, pl.ds(HD + g * _L, _L)]
                    tr = trows[b, i, pl.ds(g * _L, _L)]
                    ti = trows[b, i, pl.ds(HD + g * _L, _L)]
                    c = csrows[b, i, pl.ds(g * _L, _L)]
                    s = csrows[b, i, pl.ds(HD + g * _L, _L)]
                    dr = hr * c - hi * s - tr
                    di = hr * s + hi * c - ti
                    sq = dr * dr + di * di
                    acc = sq if acc is None else acc + sq
                rowsum = jnp.sum(acc)
                blockvec = jnp.where(lane_iota == u, rowsum, blockvec)
            outv[pl.ds(k * C + i0, _L)] = blockvec

    pltpu.sync_copy(outv, out_hbm.at[pl.ds(base, ROWS)])


def kernel(h, r, t, entity_embeddings, relation_embeddings):
    B = h.shape[0]
    R, HD = relation_embeddings.shape
    D = 2 * HD

    cs_table = pl.pallas_call(
        _trig_body,
        out_shape=jax.ShapeDtypeStruct((R, D), jnp.float32),
    )(relation_embeddings)

    ROWS = B // _NW
    C = min(128, ROWS)
    NCHUNK = ROWS // C

    mesh = plsc.VectorSubcoreMesh(core_axis_name="c", subcore_axis_name="s",
                                  num_cores=_NC, num_subcores=_NS)
    body = functools.partial(_score_body, C, D, ROWS, NCHUNK)
    score = pl.kernel(
        body,
        out_type=jax.ShapeDtypeStruct((B,), jnp.float32),
        mesh=mesh,
        compiler_params=pltpu.CompilerParams(needs_layout_passes=False),
        scratch_types=[
            pltpu.VMEM((NCHUNK, C), jnp.int32),
            pltpu.VMEM((NCHUNK, C), jnp.int32),
            pltpu.VMEM((NCHUNK, C), jnp.int32),
            pltpu.VMEM((2, C, D), jnp.float32),
            pltpu.VMEM((2, C, D), jnp.float32),
            pltpu.VMEM((2, C, D), jnp.float32),
            pltpu.VMEM((ROWS,), jnp.float32),
            pltpu.SemaphoreType.DMA,
            pltpu.SemaphoreType.DMA,
        ],
    )(h, r, t, entity_embeddings, cs_table)
    return score


# speedup vs baseline: 1.0809x; 1.0809x over previous
import functools

import jax
import jax.numpy as jnp
from jax import lax
from jax.experimental import pallas as pl
from jax.experimental.pallas import tpu as pltpu
from jax.experimental.pallas import tpu_sc as plsc

_NC = 2
_NS = 16
_NW = _NC * _NS
_L = 16


def _trig_body(rel_ref, cs_ref):
    hd = rel_ref.shape[1]
    x = rel_ref[...]
    cs_ref[:, 0:hd] = jnp.cos(x)
    cs_ref[:, hd:2 * hd] = jnp.sin(x)


def _score_body(C, D, ROWS, NCHUNK,
                h_hbm, r_hbm, t_hbm, ent_hbm, cs_hbm, out_hbm,
                hidx, ridx, tidx, hrows, trows, csrows, outv, sem0, sem1):
    HD = D // 2
    G = HD // _L
    wid = lax.axis_index("s") * _NC + lax.axis_index("c")
    base = wid * ROWS

    for k in range(NCHUNK):
        pltpu.sync_copy(h_hbm.at[pl.ds(base + k * C, C)], hidx.at[k])
        pltpu.sync_copy(t_hbm.at[pl.ds(base + k * C, C)], tidx.at[k])
        pltpu.sync_copy(r_hbm.at[pl.ds(base + k * C, C)], ridx.at[k])

    sems = (sem0, sem1)

    def issue(k):
        b = k % 2
        return (pltpu.async_copy(ent_hbm.at[hidx.at[k]], hrows.at[b], sems[b]),
                pltpu.async_copy(ent_hbm.at[tidx.at[k]], trows.at[b], sems[b]),
                pltpu.async_copy(cs_hbm.at[ridx.at[k]], csrows.at[b], sems[b]))

    pending = issue(0)
    for k in range(NCHUNK):
        b = k % 2
        nxt = issue(k + 1) if k + 1 < NCHUNK else None
        for cp in pending:
            cp.wait()
        pending = nxt

        lane_iota = lax.broadcasted_iota(jnp.int32, (_L,), 0)

        @plsc.parallel_loop(0, C // _L)
        def block_fn(jb):
            i0 = jb * _L
            blockvec = jnp.zeros((_L,), jnp.float32)
            for u in range(_L):
                i = i0 + u
                acc = None
                for g in range(G):
                    hr = hrows[b, i, pl.ds(g * _L, _L)]
                    hi = hrows[b, i, pl.ds(HD + g * _L, _L)]
                    tr = trows[b, i, pl.ds(g * _L, _L)]
                    ti = trows[b, i, pl.ds(HD + g * _L, _L)]
                    c = csrows[b, i, pl.ds(g * _L, _L)]
                    s = csrows[b, i, pl.ds(HD + g * _L, _L)]
                    dr = hr * c - hi * s - tr
                    di = hr * s + hi * c - ti
                    sq = dr * dr + di * di
                    acc = sq if acc is None else acc + sq
                rowsum = jnp.sum(acc)
                blockvec = jnp.where(lane_iota == u, rowsum, blockvec)
            outv[pl.ds(k * C + i0, _L)] = blockvec

    pltpu.sync_copy(outv, out_hbm.at[pl.ds(base, ROWS)])


def kernel(h, r, t, entity_embeddings, relation_embeddings):
    B = h.shape[0]
    R, HD = relation_embeddings.shape
    D = 2 * HD

    cs_table = pl.pallas_call(
        _trig_body,
        out_shape=jax.ShapeDtypeStruct((R, D), jnp.float32),
    )(relation_embeddings)

    ROWS = B // _NW
    C = min(128, ROWS)
    NCHUNK = ROWS // C

    mesh = plsc.VectorSubcoreMesh(core_axis_name="c", subcore_axis_name="s",
                                  num_cores=_NC, num_subcores=_NS)
    body = functools.partial(_score_body, C, D, ROWS, NCHUNK)
    score = pl.kernel(
        body,
        out_type=jax.ShapeDtypeStruct((B,), jnp.float32),
        mesh=mesh,
        compiler_params=pltpu.CompilerParams(needs_layout_passes=False),
        scratch_types=[
            pltpu.VMEM((NCHUNK, C), jnp.int32),
            pltpu.VMEM((NCHUNK, C), jnp.int32),
            pltpu.VMEM((NCHUNK, C), jnp.int32),
            pltpu.VMEM((2, C, D), jnp.float32),
            pltpu.VMEM((2, C, D), jnp.float32),
            pltpu.VMEM((2, C, D), jnp.float32),
            pltpu.VMEM((ROWS,), jnp.float32),
            pltpu.SemaphoreType.DMA,
            pltpu.SemaphoreType.DMA,
        ],
    )(h, r, t, entity_embeddings, cs_table)
    return score


# speedup vs baseline: 1.1865x; 1.0977x over previous
import functools

import jax
import jax.numpy as jnp
from jax import lax
from jax.experimental import pallas as pl
from jax.experimental.pallas import tpu as pltpu
from jax.experimental.pallas import tpu_sc as plsc

_NC = 2
_NS = 16
_NW = _NC * _NS
_L = 16


def _trig_body(rel_ref, cs_ref):
    hd = rel_ref.shape[1]
    x = rel_ref[...]
    cs_ref[:, 0:hd] = jnp.cos(x)
    cs_ref[:, hd:2 * hd] = jnp.sin(x)


def _score_body(C, D, ROWS, NCHUNK,
                h_hbm, r_hbm, t_hbm, ent_hbm, cs_hbm, out_hbm,
                hidx, ridx, tidx, hrows, trows, csrows, outv, sem0, sem1):
    HD = D // 2
    G = HD // _L
    wid = lax.axis_index("s") * _NC + lax.axis_index("c")
    base = wid * ROWS

    for k in range(NCHUNK):
        pltpu.sync_copy(h_hbm.at[pl.ds(base + k * C, C)], hidx.at[k])
        pltpu.sync_copy(t_hbm.at[pl.ds(base + k * C, C)], tidx.at[k])
        pltpu.sync_copy(r_hbm.at[pl.ds(base + k * C, C)], ridx.at[k])

    sems = (sem0, sem1)

    def issue(k):
        b = k % 2
        return (pltpu.async_copy(ent_hbm.at[hidx.at[k]], hrows.at[b], sems[b]),
                pltpu.async_copy(ent_hbm.at[tidx.at[k]], trows.at[b], sems[b]),
                pltpu.async_copy(cs_hbm.at[ridx.at[k]], csrows.at[b], sems[b]))

    pending = issue(0)
    for k in range(NCHUNK):
        b = k % 2
        nxt = issue(k + 1) if k + 1 < NCHUNK else None
        for cp in pending:
            cp.wait()
        pending = nxt

        last_lane = lax.broadcasted_iota(jnp.int32, (_L,), 0) == (_L - 1)

        @plsc.parallel_loop(0, C, unroll=2)
        def row_fn(i):
            acc = None
            for g in range(G):
                hr = hrows[b, i, pl.ds(g * _L, _L)]
                hi = hrows[b, i, pl.ds(HD + g * _L, _L)]
                tr = trows[b, i, pl.ds(g * _L, _L)]
                ti = trows[b, i, pl.ds(HD + g * _L, _L)]
                c = csrows[b, i, pl.ds(g * _L, _L)]
                s = csrows[b, i, pl.ds(HD + g * _L, _L)]
                dr = hr * c - hi * s - tr
                di = hr * s + hi * c - ti
                sq = dr * dr + di * di
                acc = sq if acc is None else acc + sq
            cum = plsc.cumsum(acc)
            plsc.store_compressed(outv.at[pl.ds(k * C + i, _L)], cum, mask=last_lane)

    pltpu.sync_copy(outv.at[pl.ds(0, ROWS)], out_hbm.at[pl.ds(base, ROWS)])


def kernel(h, r, t, entity_embeddings, relation_embeddings):
    B = h.shape[0]
    R, HD = relation_embeddings.shape
    D = 2 * HD

    cs_table = pl.pallas_call(
        _trig_body,
        out_shape=jax.ShapeDtypeStruct((R, D), jnp.float32),
    )(relation_embeddings)

    ROWS = B // _NW
    C = min(128, ROWS)
    NCHUNK = ROWS // C

    mesh = plsc.VectorSubcoreMesh(core_axis_name="c", subcore_axis_name="s",
                                  num_cores=_NC, num_subcores=_NS)
    body = functools.partial(_score_body, C, D, ROWS, NCHUNK)
    score = pl.kernel(
        body,
        out_type=jax.ShapeDtypeStruct((B,), jnp.float32),
        mesh=mesh,
        compiler_params=pltpu.CompilerParams(needs_layout_passes=False),
        scratch_types=[
            pltpu.VMEM((NCHUNK, C), jnp.int32),
            pltpu.VMEM((NCHUNK, C), jnp.int32),
            pltpu.VMEM((NCHUNK, C), jnp.int32),
            pltpu.VMEM((2, C, D), jnp.float32),
            pltpu.VMEM((2, C, D), jnp.float32),
            pltpu.VMEM((2, C, D), jnp.float32),
            pltpu.VMEM((ROWS + _L,), jnp.float32),
            pltpu.SemaphoreType.DMA,
            pltpu.SemaphoreType.DMA,
        ],
    )(h, r, t, entity_embeddings, cs_table)
    return score
